# Initial kernel scaffold; baseline (speedup 1.0000x reference)
import jax
import jax.numpy as jnp
from jax import lax
from jax.experimental import pallas as pl
from jax.experimental.pallas import tpu as pltpu
from jax.experimental.pallas import tpu_sc as plsc

NC = 2
NS = 16
NW = NC * NS
L = 16
GRP = 128

N_NODES = 10000
NPAD = 10240
E_REAL = 320000
G = -(-E_REAL // (NW * GRP))
EPAD = NW * G * GRP

IN_DIM = 128
OUT_DIM = 128
FEAT_DIM = 64


def _proj_body(h_ref, wfc_ref, wdf_ref, z_ref, u_ref):
    h = h_ref[...]
    wfc_t = wfc_ref[...].T
    z_ref[...] = jnp.dot(h, wfc_t, preferred_element_type=jnp.float32)
    q = jnp.dot(wfc_t, wdf_ref[...].T, preferred_element_type=jnp.float32)
    u_ref[...] = jnp.dot(h, q, preferred_element_type=jnp.float32)


def _proj(h, w_fc, w_dstfeat):
    n = h.shape[0]
    return pl.pallas_call(
        _proj_body,
        out_shape=(
            jax.ShapeDtypeStruct((n, OUT_DIM), jnp.float32),
            jax.ShapeDtypeStruct((n, FEAT_DIM), jnp.float32),
        ),
    )(h, w_fc, w_dstfeat)


def _dinv_body(dp_ref, dinv_ref):
    den = jnp.sum(dp_ref[...], axis=0)
    dinv_ref[...] = 1.0 / jnp.maximum(den, 1e-30)


def _dinv(den_parts):
    return pl.pallas_call(
        _dinv_body,
        out_shape=jax.ShapeDtypeStruct((NPAD,), jnp.float32),
    )(den_parts)


def _add_body(a_ref, b_ref, o_ref):
    o_ref[...] = a_ref[...] + b_ref[...]


def _final_add(a, b):
    return pl.pallas_call(
        _add_body,
        out_shape=jax.ShapeDtypeStruct(a.shape, jnp.float32),
    )(a, b)


def _sc_mesh():
    return plsc.VectorSubcoreMesh(
        core_axis_name="c", subcore_axis_name="s", num_cores=NC, num_subcores=NS
    )


def _edge_body(u_hbm, feat_hbm, src_hbm, dst_hbm, ee_hbm, den_hbm,
               src_t, dst_t, u_buf, f_buf, ee_t, den_t, win_t, sem1, sem2):
    c = lax.axis_index("c")
    s = lax.axis_index("s")
    wid = s * NC + c
    iota = lax.iota(jnp.int32, L)

    pltpu.sync_copy(src_hbm.at[wid], src_t)
    pltpu.sync_copy(dst_hbm.at[wid], dst_t)

    zero16 = jnp.zeros((L,), jnp.float32)

    def _zero(i, _):
        den_t[pl.ds(i * L, L)] = zero16
        return 0

    lax.fori_loop(0, NPAD // L, _zero, 0)

    def _group(g, _):
        cp1 = pltpu.async_copy(u_hbm.at[src_t.at[g]], u_buf, sem1)
        cp2 = pltpu.async_copy(feat_hbm.at[dst_t.at[g]], f_buf, sem2)
        cp1.wait()
        cp2.wait()
        for v in range(GRP // L):
            rows = v * L + iota
            acc = jnp.zeros((L,), jnp.float32)

            def _dot(kk, acc):
                for k8 in range(8):
                    colk = jnp.full((L,), 0, jnp.int32) + (kk * 8 + k8)
                    uv = plsc.load_gather(u_buf, [rows, colk])
                    fv = plsc.load_gather(f_buf, [rows, colk])
                    acc = acc + uv * fv
                return acc

            acc = lax.fori_loop(0, FEAT_DIM // 8, _dot, acc)
            ee_v = jnp.exp(jnp.minimum(acc, 80.0))
            geid = (wid * G + g) * GRP + v * L + iota
            ee_v = jnp.where(geid < E_REAL, ee_v, 0.0)
            ee_t[g, pl.ds(v * L, L)] = ee_v

            dstv = dst_t[g, pl.ds(v * L, L)]

            def _cond(pend):
                return jnp.any(pend)

            def _round(pend):
                plsc.store_scatter(win_t, [dstv], iota, mask=pend)
                back = plsc.load_gather(win_t, [dstv])
                win = (back == iota) & pend
                cur = plsc.load_gather(den_t, [dstv])
                plsc.store_scatter(den_t, [dstv], cur + ee_v, mask=win)
                return pend & jnp.logical_not(win)

            lax.while_loop(_cond, _round, jnp.ones((L,), jnp.bool_))
        return 0

    lax.fori_loop(0, G, _group, 0)

    pltpu.sync_copy(ee_t, ee_hbm.at[wid])
    pltpu.sync_copy(den_t, den_hbm.at[wid])


def _edge_pass(u, feat, src_r, dst_r):
    kern = pl.kernel(
        _edge_body,
        out_type=(
            jax.ShapeDtypeStruct((NW, G, GRP), jnp.float32),
            jax.ShapeDtypeStruct((NW, NPAD), jnp.float32),
        ),
        mesh=_sc_mesh(),
        scratch_types=[
            pltpu.VMEM((G, GRP), jnp.int32),
            pltpu.VMEM((G, GRP), jnp.int32),
            pltpu.VMEM((GRP, FEAT_DIM), jnp.float32),
            pltpu.VMEM((GRP, FEAT_DIM), jnp.float32),
            pltpu.VMEM((G, GRP), jnp.float32),
            pltpu.VMEM((NPAD,), jnp.float32),
            pltpu.VMEM((NPAD,), jnp.int32),
            pltpu.SemaphoreType.DMA,
            pltpu.SemaphoreType.DMA,
        ],
    )
    return kern(u, feat, src_r, dst_r)


def _scatter_body(z_hbm, ee_hbm, src_hbm, dst_hbm, dinv_hbm, out_hbm,
                  src_t, dst_t, alpha_t, dinv_t, z_buf, acc_sh, sem1):
    c = lax.axis_index("c")
    s = lax.axis_index("s")
    wid = s * NC + c
    iota = lax.iota(jnp.int32, L)
    rows_per_w = NPAD // NS

    pltpu.sync_copy(src_hbm.at[wid], src_t)
    pltpu.sync_copy(dst_hbm.at[wid], dst_t)
    pltpu.sync_copy(dinv_hbm, dinv_t)
    pltpu.sync_copy(ee_hbm.at[wid], alpha_t)

    def _alpha(g, _):
        for v in range(GRP // L):
            dstv = dst_t[g, pl.ds(v * L, L)]
            di = plsc.load_gather(dinv_t, [dstv])
            alpha_t[g, pl.ds(v * L, L)] = alpha_t[g, pl.ds(v * L, L)] * di
        return 0

    lax.fori_loop(0, G, _alpha, 0)

    zero16 = jnp.zeros((L,), jnp.float32)

    def _zrow(i, _):
        z_buf[i % GRP, pl.ds((i // GRP) * L, L)] = zero16
        return 0

    lax.fori_loop(0, GRP * (OUT_DIM // L), _zrow, 0)
    for i in range(rows_per_w // GRP):
        pltpu.sync_copy(z_buf, acc_sh.at[pl.ds(s * rows_per_w + i * GRP, GRP)])
    plsc.subcore_barrier()

    def _group(g, _):
        pltpu.async_copy(z_hbm.at[src_t.at[g]], z_buf, sem1).wait()
        for v in range(GRP // L):
            av = alpha_t[g, pl.ds(v * L, L)]
            rows = v * L + iota

            def _scale(kk, _):
                for k8 in range(8):
                    colk = jnp.full((L,), 0, jnp.int32) + (kk * 8 + k8)
                    val = plsc.load_gather(z_buf, [rows, colk])
                    plsc.store_scatter(z_buf, [rows, colk], val * av)
                return 0

            lax.fori_loop(0, OUT_DIM // 8, _scale, 0)
        pltpu.sync_copy(z_buf, acc_sh.at[dst_t.at[g]], add=True)
        return 0

    lax.fori_loop(0, G, _group, 0)
    plsc.subcore_barrier()

    pltpu.sync_copy(
        acc_sh.at[pl.ds(s * rows_per_w, rows_per_w)],
        out_hbm.at[c, pl.ds(s * rows_per_w, rows_per_w)],
    )


def _scatter_pass(z, ee, src_r, dst_r, dinv):
    kern = pl.kernel(
        _scatter_body,
        out_type=jax.ShapeDtypeStruct((NC, NPAD, OUT_DIM), jnp.float32),
        mesh=_sc_mesh(),
        scratch_types=[
            pltpu.VMEM((G, GRP), jnp.int32),
            pltpu.VMEM((G, GRP), jnp.int32),
            pltpu.VMEM((G, GRP), jnp.float32),
            pltpu.VMEM((NPAD,), jnp.float32),
            pltpu.VMEM((GRP, OUT_DIM), jnp.float32),
            pltpu.VMEM_SHARED((NPAD, OUT_DIM), jnp.float32),
            pltpu.SemaphoreType.DMA,
        ],
    )
    return kern(z, ee, src_r, dst_r, dinv)


def kernel(h, feat, edge_index, W_fc, W_dstfeat):
    src = edge_index[0]
    dst = edge_index[1]
    pad = EPAD - E_REAL
    src_r = jnp.pad(src, (0, pad)).reshape(NW, G, GRP)
    dst_r = jnp.pad(dst, (0, pad)).reshape(NW, G, GRP)

    z, u = _proj(h, W_fc, W_dstfeat)
    ee, den_parts = _edge_pass(u, feat, src_r, dst_r)
    dinv = _dinv(den_parts)
    out_parts = _scatter_pass(z, ee, src_r, dst_r, dinv)
    out = _final_add(out_parts[0, :N_NODES], out_parts[1, :N_NODES])
    return out


# speedup vs baseline: 6.7555x; 6.7555x over previous
import jax
import jax.numpy as jnp
from jax import lax
from jax.experimental import pallas as pl
from jax.experimental.pallas import tpu as pltpu
from jax.experimental.pallas import tpu_sc as plsc

NC = 2
NS = 16
NW = NC * NS
L = 16
GRP = 128
GRP2 = 64

N_NODES = 10000
NPAD = 10240
E_REAL = 320000
G = 80
G2 = G * GRP // GRP2
EPAD = NW * G * GRP

IN_DIM = 128
OUT_DIM = 128
FEAT_DIM = 64

_HI = lax.Precision.HIGHEST


def _proj_body(h_ref, feat_ref, wfc_ref, wdf_ref, z_ref, d_ref):
    z_ref[...] = jnp.dot(h_ref[...], wfc_ref[...].T,
                         preferred_element_type=jnp.float32)
    d_ref[...] = jnp.dot(feat_ref[...], wdf_ref[...].T,
                         preferred_element_type=jnp.float32)


def _proj(h, feat, w_fc, w_dstfeat):
    n = h.shape[0]
    return pl.pallas_call(
        _proj_body,
        out_shape=(
            jax.ShapeDtypeStruct((n, OUT_DIM), jnp.float32),
            jax.ShapeDtypeStruct((n, OUT_DIM), jnp.float32),
        ),
    )(h, feat, w_fc, w_dstfeat)


def _combine_body(p_ref, dp_ref, o_ref):
    den = jnp.sum(dp_ref[...], axis=0)[:N_NODES]
    dinv = 1.0 / jnp.maximum(den, 1e-30)
    o_ref[...] = (p_ref[0] + p_ref[1]) * dinv[:, None]


def _combine(parts, den_parts):
    return pl.pallas_call(
        _combine_body,
        out_shape=jax.ShapeDtypeStruct((N_NODES, OUT_DIM), jnp.float32),
    )(parts, den_parts)


_GATHER_DNUMS = lax.GatherDimensionNumbers(
    offset_dims=(), collapsed_slice_dims=(0,), start_index_map=(0,)
)


def _lane_shuffle(x, idx):
    return lax.gather(
        x, idx[:, None], dimension_numbers=_GATHER_DNUMS, slice_sizes=(1,),
        mode=lax.GatherScatterMode.PROMISE_IN_BOUNDS,
    )


def _sc_mesh():
    return plsc.VectorSubcoreMesh(
        core_axis_name="c", subcore_axis_name="s", num_cores=NC, num_subcores=NS
    )


_SC_PARAMS = pltpu.CompilerParams(
    needs_layout_passes=False, use_tc_tiling_on_sc=False
)


def _edge_body(u_hbm, feat_hbm, src_hbm, dst_hbm, ee_hbm, den_hbm,
               src_t, dst_t, u0, f0, u1, f1, ee_t, den_t,
               su0, sf0, su1, sf1):
    c = lax.axis_index("c")
    s = lax.axis_index("s")
    wid = s * NC + c
    iota = lax.iota(jnp.int32, L)

    pltpu.sync_copy(src_hbm.at[wid], src_t)
    pltpu.sync_copy(dst_hbm.at[wid], dst_t)

    zero16 = jnp.zeros((L,), jnp.float32)

    def _zero(i, _):
        den_t[pl.ds(i * L, L)] = zero16
        return 0

    lax.fori_loop(0, NPAD // L, _zero, 0)

    slots = ((u0, f0, su0, sf0), (u1, f1, su1, sf1))
    for sl in range(2):
        ub, fb, su, sf = slots[sl]
        pltpu.async_copy(u_hbm.at[src_t.at[sl]], ub, su)
        pltpu.async_copy(feat_hbm.at[dst_t.at[sl]], fb, sf)

    def _do_group(g, ub, fb):
        def _vblock(v, _):
            e_vec = jnp.zeros((L,), jnp.float32)
            for j in range(L):
                row = v * L + j
                acc = jnp.zeros((L,), jnp.float32)
                for k in range(OUT_DIM // L):
                    acc = acc + (ub[row, pl.ds(k * L, L)]
                                 * fb[row, pl.ds(k * L, L)])
                for sh in (8, 4, 2, 1):
                    acc = acc + _lane_shuffle(acc, iota ^ sh)
                e_vec = jnp.where(iota == j, acc, e_vec)
            ee_v = jnp.exp(jnp.minimum(e_vec, 80.0))
            geid = (wid * G + g) * GRP + v * L + iota
            ee_v = jnp.where(geid < E_REAL, ee_v, 0.0)
            ee_t[g, pl.ds(v * L, L)] = ee_v

            dstv = dst_t[g, pl.ds(v * L, L)]
            total = ee_v
            haslater = iota < 0
            for o in range(1, L):
                pidx = jnp.maximum(iota - o, 0)
                m_prev = (_lane_shuffle(dstv, pidx) == dstv) & (iota >= o)
                total = total + jnp.where(
                    m_prev, _lane_shuffle(ee_v, pidx), 0.0)
                nidx = jnp.minimum(iota + o, L - 1)
                m_next = ((_lane_shuffle(dstv, nidx) == dstv)
                          & (iota <= L - 1 - o))
                haslater = haslater | m_next
            cur = plsc.load_gather(den_t, [dstv])
            plsc.store_scatter(den_t, [dstv], cur + total,
                               mask=jnp.logical_not(haslater))
            return 0

        lax.fori_loop(0, GRP // L, _vblock, 0)

    def _pair(gg, _):
        for sl in range(2):
            ub, fb, su, sf = slots[sl]
            g = gg * 2 + sl
            pltpu.make_async_copy(u_hbm.at[src_t.at[g]], ub, su).wait()
            pltpu.make_async_copy(feat_hbm.at[dst_t.at[g]], fb, sf).wait()
            _do_group(g, ub, fb)
            gn = jnp.minimum(g + 2, G - 1)
            pltpu.async_copy(u_hbm.at[src_t.at[gn]], ub, su)
            pltpu.async_copy(feat_hbm.at[dst_t.at[gn]], fb, sf)
        return 0

    lax.fori_loop(0, G // 2, _pair, 0)
    for sl in range(2):
        ub, fb, su, sf = slots[sl]
        pltpu.make_async_copy(u_hbm.at[src_t.at[G - 1]], ub, su).wait()
        pltpu.make_async_copy(feat_hbm.at[dst_t.at[G - 1]], fb, sf).wait()

    pltpu.sync_copy(ee_t, ee_hbm.at[wid])
    pltpu.sync_copy(den_t, den_hbm.at[wid])


def _edge_pass(u, feat, src_r, dst_r):
    kern = pl.kernel(
        _edge_body,
        out_type=(
            jax.ShapeDtypeStruct((NW, G, GRP), jnp.float32),
            jax.ShapeDtypeStruct((NW, NPAD), jnp.float32),
        ),
        mesh=_sc_mesh(),
        compiler_params=_SC_PARAMS,
        scratch_types=[
            pltpu.VMEM((G, GRP), jnp.int32),
            pltpu.VMEM((G, GRP), jnp.int32),
            pltpu.VMEM((GRP, OUT_DIM), jnp.float32),
            pltpu.VMEM((GRP, OUT_DIM), jnp.float32),
            pltpu.VMEM((GRP, OUT_DIM), jnp.float32),
            pltpu.VMEM((GRP, OUT_DIM), jnp.float32),
            pltpu.VMEM((G, GRP), jnp.float32),
            pltpu.VMEM((NPAD,), jnp.float32),
            pltpu.SemaphoreType.DMA,
            pltpu.SemaphoreType.DMA,
            pltpu.SemaphoreType.DMA,
            pltpu.SemaphoreType.DMA,
        ],
    )
    return kern(u, feat, src_r, dst_r)


def _scatter_body(z_hbm, ee_hbm, src_hbm, dst_hbm, out_hbm,
                  src_t, dst_t, ee_t, z0, z1, acc_sh,
                  sg0, sg1, ss0, ss1):
    c = lax.axis_index("c")
    s = lax.axis_index("s")
    wid = s * NC + c
    rows_per_w = N_NODES // NS

    pltpu.sync_copy(src_hbm.at[wid], src_t)
    pltpu.sync_copy(dst_hbm.at[wid], dst_t)
    pltpu.sync_copy(ee_hbm.at[wid], ee_t)

    zero16 = jnp.zeros((L,), jnp.float32)

    def _zrow(i, _):
        z0[i % GRP2, pl.ds((i // GRP2) * L, L)] = zero16
        z1[i % GRP2, pl.ds((i // GRP2) * L, L)] = zero16
        return 0

    lax.fori_loop(0, GRP2 * (OUT_DIM // L), _zrow, 0)
    for i in range(-(-rows_per_w // GRP2)):
        base = i * GRP2
        nrows = min(GRP2, rows_per_w - base)
        pltpu.sync_copy(
            z0.at[pl.ds(0, nrows)],
            acc_sh.at[pl.ds(s * rows_per_w + base, nrows)],
        )
    plsc.subcore_barrier()

    slots = ((z0, sg0, ss0), (z1, sg1, ss1))
    for sl in range(2):
        zb, sg, ss = slots[sl]
        pltpu.async_copy(z_hbm.at[src_t.at[sl]], zb, sg)

    def _do_group(g, zb):
        def _vblock(v, _):
            av = ee_t[g, pl.ds(v * L, L)]
            for j in range(L):
                row = v * L + j
                a_s = jnp.full((L,), av[j], jnp.float32)
                for k in range(OUT_DIM // L):
                    zb[row, pl.ds(k * L, L)] = zb[row, pl.ds(k * L, L)] * a_s
            return 0

        lax.fori_loop(0, GRP2 // L, _vblock, 0)

    def _pair(gg, _):
        for sl in range(2):
            zb, sg, ss = slots[sl]
            g = gg * 2 + sl
            pltpu.make_async_copy(z_hbm.at[src_t.at[g]], zb, sg).wait()
            _do_group(g, zb)
            pltpu.async_copy(zb, acc_sh.at[dst_t.at[g]], ss, add=True).wait()
            gn = jnp.minimum(g + 2, G2 - 1)
            pltpu.async_copy(z_hbm.at[src_t.at[gn]], zb, sg)
        return 0

    lax.fori_loop(0, G2 // 2, _pair, 0)
    for sl in range(2):
        zb, sg, ss = slots[sl]
        pltpu.make_async_copy(z_hbm.at[src_t.at[G2 - 1]], zb, sg).wait()
    plsc.subcore_barrier()

    pltpu.sync_copy(
        acc_sh.at[pl.ds(s * rows_per_w, rows_per_w)],
        out_hbm.at[c, pl.ds(s * rows_per_w, rows_per_w)],
    )


def _scatter_pass(z, ee2, src_r2, dst_r2):
    kern = pl.kernel(
        _scatter_body,
        out_type=jax.ShapeDtypeStruct((NC, N_NODES, OUT_DIM), jnp.float32),
        mesh=_sc_mesh(),
        compiler_params=_SC_PARAMS,
        scratch_types=[
            pltpu.VMEM((G2, GRP2), jnp.int32),
            pltpu.VMEM((G2, GRP2), jnp.int32),
            pltpu.VMEM((G2, GRP2), jnp.float32),
            pltpu.VMEM((GRP2, OUT_DIM), jnp.float32),
            pltpu.VMEM((GRP2, OUT_DIM), jnp.float32),
            pltpu.VMEM_SHARED((N_NODES, OUT_DIM), jnp.float32),
            pltpu.SemaphoreType.DMA,
            pltpu.SemaphoreType.DMA,
            pltpu.SemaphoreType.DMA,
            pltpu.SemaphoreType.DMA,
        ],
    )
    return kern(z, ee2, src_r2, dst_r2)


def kernel(h, feat, edge_index, W_fc, W_dstfeat):
    src = edge_index[0]
    dst = edge_index[1]
    pad = EPAD - E_REAL
    src_p = jnp.pad(src, (0, pad))
    dst_p = jnp.pad(dst, (0, pad))

    z, dd = _proj(h, feat, W_fc, W_dstfeat)
    ee, den_parts = _edge_pass(z, dd,
                               src_p.reshape(NW, G, GRP),
                               dst_p.reshape(NW, G, GRP))
    parts = _scatter_pass(z, ee.reshape(NW, G2, GRP2),
                          src_p.reshape(NW, G2, GRP2),
                          dst_p.reshape(NW, G2, GRP2))
    return _combine(parts, den_parts)


# speedup vs baseline: 11.6779x; 1.7287x over previous
import jax
import jax.numpy as jnp
from jax import lax
from jax.experimental import pallas as pl
from jax.experimental.pallas import tpu as pltpu
from jax.experimental.pallas import tpu_sc as plsc

NC = 2
NS = 16
NW = NC * NS
L = 16
GRP = 16

N_NODES = 10000
E_REAL = 320000
NG = 636
EPAD = NW * NG * GRP

OUT_DIM = 128


def _proj_body(h_ref, feat_ref, wfc_ref, wdf_ref, z_ref, d_ref):
    z_ref[...] = jnp.dot(h_ref[...], wfc_ref[...].T,
                         preferred_element_type=jnp.float32)
    d_ref[...] = jnp.dot(feat_ref[...], wdf_ref[...].T,
                         preferred_element_type=jnp.float32)


def _proj(h, feat, w_fc, w_dstfeat):
    n = h.shape[0]
    return pl.pallas_call(
        _proj_body,
        out_shape=(
            jax.ShapeDtypeStruct((n, OUT_DIM), jnp.float32),
            jax.ShapeDtypeStruct((n, OUT_DIM), jnp.float32),
        ),
    )(h, feat, w_fc, w_dstfeat)


def _combine_body(p_ref, dp_ref, o_ref):
    den = jnp.sum(dp_ref[...], axis=0)
    dinv = 1.0 / jnp.maximum(den, 1e-30)
    o_ref[...] = (p_ref[0] + p_ref[1]) * dinv[:, None]


def _combine(parts, den_parts):
    return pl.pallas_call(
        _combine_body,
        out_shape=jax.ShapeDtypeStruct((N_NODES, OUT_DIM), jnp.float32),
    )(parts, den_parts)


_GATHER_DNUMS = lax.GatherDimensionNumbers(
    offset_dims=(), collapsed_slice_dims=(0,), start_index_map=(0,)
)


def _lane_shuffle(x, idx):
    return lax.gather(
        x, idx[:, None], dimension_numbers=_GATHER_DNUMS, slice_sizes=(1,),
        mode=lax.GatherScatterMode.PROMISE_IN_BOUNDS,
    )


def _sc_mesh():
    return plsc.VectorSubcoreMesh(
        core_axis_name="c", subcore_axis_name="s", num_cores=NC, num_subcores=NS
    )


_SC_PARAMS = pltpu.CompilerParams(
    needs_layout_passes=False, use_tc_tiling_on_sc=False
)


def _fused_body(z_hbm, d_hbm, src_hbm, dst_hbm, den_hbm, out_hbm,
                src_t, dst_t, den_t, z0, z1, z2, d0, d1, d2, acc_sh,
                sz0, sz1, sz2, sd0, sd1, sd2, ss0, ss1, ss2):
    c = lax.axis_index("c")
    s = lax.axis_index("s")
    wid = s * NC + c
    iota = lax.iota(jnp.int32, L)
    rows_per_w = N_NODES // NS

    pltpu.sync_copy(src_hbm.at[wid], src_t)
    pltpu.sync_copy(dst_hbm.at[wid], dst_t)

    zero16 = jnp.zeros((L,), jnp.float32)

    def _zero(i, _):
        den_t[pl.ds(i * L, L)] = zero16
        return 0

    lax.fori_loop(0, N_NODES // L, _zero, 0)

    def _zrow(i, _):
        z0[i % GRP, pl.ds((i // GRP) * L, L)] = zero16
        z2[i % GRP, pl.ds((i // GRP) * L, L)] = zero16
        return 0

    lax.fori_loop(0, GRP * (OUT_DIM // L), _zrow, 0)
    for i in range(-(-rows_per_w // GRP)):
        base = i * GRP
        nrows = min(GRP, rows_per_w - base)
        pltpu.sync_copy(
            z0.at[pl.ds(0, nrows)],
            acc_sh.at[pl.ds(s * rows_per_w + base, nrows)],
        )
    plsc.subcore_barrier()

    zslots = ((z0, sz0, ss0), (z1, sz1, ss1), (z2, sz2, ss2))
    dslots = ((d0, sd0), (d1, sd1), (d2, sd2))

    pltpu.async_copy(z2, acc_sh.at[dst_t.at[0]], ss2, add=True)
    for g in range(2):
        zb, sz, _ = zslots[g]
        pltpu.async_copy(z_hbm.at[src_t.at[g]], zb, sz)
    for g in range(3):
        db, sd = dslots[g]
        pltpu.async_copy(d_hbm.at[dst_t.at[g]], db, sd)

    def _do_group(g, zb, db):
        def _vblock(v, _):
            e_vec = jnp.zeros((L,), jnp.float32)
            for j in range(L):
                row = v * L + j
                acc = jnp.zeros((L,), jnp.float32)
                for k in range(OUT_DIM // L):
                    acc = acc + (zb[row, pl.ds(k * L, L)]
                                 * db[row, pl.ds(k * L, L)])
                for sh in (8, 4, 2, 1):
                    acc = acc + _lane_shuffle(acc, iota ^ sh)
                e_vec = jnp.where(iota == j, acc, e_vec)
            ee_v = jnp.exp(jnp.minimum(e_vec, 80.0))
            geid = (wid * NG + g) * GRP + v * L + iota
            ee_v = jnp.where(geid < E_REAL, ee_v, 0.0)

            dstv = dst_t[g, pl.ds(v * L, L)]
            total = ee_v
            haslater = iota < 0
            for o in range(1, L):
                pidx = jnp.maximum(iota - o, 0)
                m_prev = (_lane_shuffle(dstv, pidx) == dstv) & (iota >= o)
                total = total + jnp.where(
                    m_prev, _lane_shuffle(ee_v, pidx), 0.0)
                nidx = jnp.minimum(iota + o, L - 1)
                m_next = ((_lane_shuffle(dstv, nidx) == dstv)
                          & (iota <= L - 1 - o))
                haslater = haslater | m_next
            cur = plsc.load_gather(den_t, [dstv])
            plsc.store_scatter(den_t, [dstv], cur + total,
                               mask=jnp.logical_not(haslater))

            for j in range(L):
                row = v * L + j
                a_s = jnp.full((L,), ee_v[j], jnp.float32)
                for k in range(OUT_DIM // L):
                    zb[row, pl.ds(k * L, L)] = zb[row, pl.ds(k * L, L)] * a_s
            return 0

        lax.fori_loop(0, GRP // L, _vblock, 0)

    def _three(gg, _):
        for sl in range(3):
            g = gg * 3 + sl
            zb, sz, ss = zslots[sl]
            db, sd = dslots[sl]
            pltpu.make_async_copy(z_hbm.at[src_t.at[g]], zb, sz).wait()
            pltpu.make_async_copy(d_hbm.at[dst_t.at[g]], db, sd).wait()
            _do_group(g, zb, db)
            pltpu.async_copy(zb, acc_sh.at[dst_t.at[g]], ss, add=True)
            gd = jnp.minimum(g + 3, NG - 1)
            pltpu.async_copy(d_hbm.at[dst_t.at[gd]], db, sd)
            zb2, sz2_, ss2_ = zslots[(sl + 2) % 3]
            gp = jnp.maximum(g - 1, 0)
            pltpu.make_async_copy(zb2, acc_sh.at[dst_t.at[gp]], ss2_).wait()
            gz = jnp.minimum(g + 2, NG - 1)
            pltpu.async_copy(z_hbm.at[src_t.at[gz]], zb2, sz2_)
        return 0

    lax.fori_loop(0, NG // 3, _three, 0)

    for sl in range(2):
        zb, sz, _ = zslots[sl]
        pltpu.make_async_copy(z_hbm.at[src_t.at[NG - 1]], zb, sz).wait()
    for sl in range(3):
        db, sd = dslots[sl]
        pltpu.make_async_copy(d_hbm.at[dst_t.at[NG - 1]], db, sd).wait()
    zb, _, ss = zslots[(NG - 1) % 3]
    pltpu.make_async_copy(zb, acc_sh.at[dst_t.at[NG - 1]], ss).wait()
    plsc.subcore_barrier()

    pltpu.sync_copy(
        acc_sh.at[pl.ds(s * rows_per_w, rows_per_w)],
        out_hbm.at[c, pl.ds(s * rows_per_w, rows_per_w)],
    )
    pltpu.sync_copy(den_t, den_hbm.at[wid])


def _fused_pass(z, d, src_r, dst_r):
    kern = pl.kernel(
        _fused_body,
        out_type=(
            jax.ShapeDtypeStruct((NW, N_NODES), jnp.float32),
            jax.ShapeDtypeStruct((NC, N_NODES, OUT_DIM), jnp.float32),
        ),
        mesh=_sc_mesh(),
        compiler_params=_SC_PARAMS,
        scratch_types=[
            pltpu.VMEM((NG, GRP), jnp.int32),
            pltpu.VMEM((NG, GRP), jnp.int32),
            pltpu.VMEM((N_NODES,), jnp.float32),
            pltpu.VMEM((GRP, OUT_DIM), jnp.float32),
            pltpu.VMEM((GRP, OUT_DIM), jnp.float32),
            pltpu.VMEM((GRP, OUT_DIM), jnp.float32),
            pltpu.VMEM((GRP, OUT_DIM), jnp.float32),
            pltpu.VMEM((GRP, OUT_DIM), jnp.float32),
            pltpu.VMEM((GRP, OUT_DIM), jnp.float32),
            pltpu.VMEM_SHARED((N_NODES, OUT_DIM), jnp.float32),
        ] + [pltpu.SemaphoreType.DMA] * 9,
    )
    return kern(z, d, src_r, dst_r)


def kernel(h, feat, edge_index, W_fc, W_dstfeat):
    src = edge_index[0]
    dst = edge_index[1]
    pad = EPAD - E_REAL
    src_r = jnp.pad(src, (0, pad)).reshape(NW, NG, GRP)
    dst_r = jnp.pad(dst, (0, pad)).reshape(NW, NG, GRP)

    z, d = _proj(h, feat, W_fc, W_dstfeat)
    den_parts, parts = _fused_pass(z, d, src_r, dst_r)
    return _combine(parts, den_parts)


# speedup vs baseline: 12.5379x; 1.0736x over previous
import jax
import jax.numpy as jnp
from jax import lax
from jax.experimental import pallas as pl
from jax.experimental.pallas import tpu as pltpu
from jax.experimental.pallas import tpu_sc as plsc

NC = 2
NS = 16
NW = NC * NS
L = 16
GRP = 32

N_NODES = 10000
E_REAL = 320000
NG = 318
EPAD = NW * NG * GRP

OUT_DIM = 128


def _proj_body(h_ref, feat_ref, wfc_ref, wdf_ref, z_ref, d_ref):
    z_ref[...] = jnp.dot(h_ref[...], wfc_ref[...].T,
                         preferred_element_type=jnp.float32)
    d_ref[...] = jnp.dot(feat_ref[...], wdf_ref[...].T,
                         preferred_element_type=jnp.float32)


def _proj(h, feat, w_fc, w_dstfeat):
    n = h.shape[0]
    return pl.pallas_call(
        _proj_body,
        out_shape=(
            jax.ShapeDtypeStruct((n, OUT_DIM), jnp.float32),
            jax.ShapeDtypeStruct((n, OUT_DIM), jnp.float32),
        ),
    )(h, feat, w_fc, w_dstfeat)


def _combine_body(p_ref, dp_ref, o_ref):
    den = jnp.sum(dp_ref[...], axis=0)
    dinv = 1.0 / jnp.maximum(den, 1e-30)
    o_ref[...] = (p_ref[0] + p_ref[1]) * dinv[:, None]


def _combine(parts, den_parts):
    return pl.pallas_call(
        _combine_body,
        out_shape=jax.ShapeDtypeStruct((N_NODES, OUT_DIM), jnp.float32),
    )(parts, den_parts)


_GATHER_DNUMS = lax.GatherDimensionNumbers(
    offset_dims=(), collapsed_slice_dims=(0,), start_index_map=(0,)
)


def _lane_shuffle(x, idx):
    return lax.gather(
        x, idx[:, None], dimension_numbers=_GATHER_DNUMS, slice_sizes=(1,),
        mode=lax.GatherScatterMode.PROMISE_IN_BOUNDS,
    )


def _sc_mesh():
    return plsc.VectorSubcoreMesh(
        core_axis_name="c", subcore_axis_name="s", num_cores=NC, num_subcores=NS
    )


_SC_PARAMS = pltpu.CompilerParams(
    needs_layout_passes=False, use_tc_tiling_on_sc=False
)


def _fused_body(z_hbm, d_hbm, src_hbm, dst_hbm, den_hbm, out_hbm,
                src_t, dst_t, den_t, z0, z1, d0, d1, acc_sh,
                sz0, sz1, sd0, sd1, ss0, ss1):
    c = lax.axis_index("c")
    s = lax.axis_index("s")
    wid = s * NC + c
    iota = lax.iota(jnp.int32, L)
    rows_per_w = N_NODES // NS

    pltpu.sync_copy(src_hbm.at[wid], src_t)
    pltpu.sync_copy(dst_hbm.at[wid], dst_t)

    zero16 = jnp.zeros((L,), jnp.float32)

    def _zero(i, _):
        den_t[pl.ds(i * L, L)] = zero16
        return 0

    lax.fori_loop(0, N_NODES // L, _zero, 0)

    def _zrow(i, _):
        z0[i % GRP, pl.ds((i // GRP) * L, L)] = zero16
        return 0

    lax.fori_loop(0, GRP * (OUT_DIM // L), _zrow, 0)
    for i in range(-(-rows_per_w // GRP)):
        base = i * GRP
        nrows = min(GRP, rows_per_w - base)
        pltpu.sync_copy(
            z0.at[pl.ds(0, nrows)],
            acc_sh.at[pl.ds(s * rows_per_w + base, nrows)],
        )
    plsc.subcore_barrier()

    zslots = ((z0, sz0, ss0), (z1, sz1, ss1))
    dslots = ((d0, sd0), (d1, sd1))

    for g in range(2):
        zb, sz, _ = zslots[g]
        db, sd = dslots[g]
        pltpu.async_copy(z_hbm.at[src_t.at[g]], zb, sz)
        pltpu.async_copy(d_hbm.at[dst_t.at[g]], db, sd)

    def _do_group(g, zb, db):
        def _vblock(v, _):
            e_vec = jnp.zeros((L,), jnp.float32)
            for j in range(L):
                row = v * L + j
                acc = jnp.zeros((L,), jnp.float32)
                for k in range(OUT_DIM // L):
                    acc = acc + (zb[row, pl.ds(k * L, L)]
                                 * db[row, pl.ds(k * L, L)])
                for sh in (8, 4, 2, 1):
                    acc = acc + _lane_shuffle(acc, iota ^ sh)
                e_vec = jnp.where(iota == j, acc, e_vec)
            ee_v = jnp.exp(jnp.minimum(e_vec, 80.0))
            geid = (wid * NG + g) * GRP + v * L + iota
            ee_v = jnp.where(geid < E_REAL, ee_v, 0.0)

            dstv = dst_t[g, pl.ds(v * L, L)]
            total = ee_v
            haslater = iota < 0
            for o in range(1, L):
                pidx = jnp.maximum(iota - o, 0)
                m_prev = (_lane_shuffle(dstv, pidx) == dstv) & (iota >= o)
                total = total + jnp.where(
                    m_prev, _lane_shuffle(ee_v, pidx), 0.0)
                nidx = jnp.minimum(iota + o, L - 1)
                m_next = ((_lane_shuffle(dstv, nidx) == dstv)
                          & (iota <= L - 1 - o))
                haslater = haslater | m_next
            cur = plsc.load_gather(den_t, [dstv])
            plsc.store_scatter(den_t, [dstv], cur + total,
                               mask=jnp.logical_not(haslater))

            for j in range(L):
                row = v * L + j
                a_s = jnp.full((L,), ee_v[j], jnp.float32)
                for k in range(OUT_DIM // L):
                    zb[row, pl.ds(k * L, L)] = zb[row, pl.ds(k * L, L)] * a_s
            return 0

        lax.fori_loop(0, GRP // L, _vblock, 0)

    def _pair(gg, _):
        for sl in range(2):
            g = gg * 2 + sl
            zb, sz, ss = zslots[sl]
            db, sd = dslots[sl]
            pltpu.make_async_copy(z_hbm.at[src_t.at[g]], zb, sz).wait()
            pltpu.make_async_copy(d_hbm.at[dst_t.at[g]], db, sd).wait()
            _do_group(g, zb, db)
            pltpu.async_copy(zb, acc_sh.at[dst_t.at[g]], ss, add=True).wait()
            gn = jnp.minimum(g + 2, NG - 1)
            pltpu.async_copy(z_hbm.at[src_t.at[gn]], zb, sz)
            pltpu.async_copy(d_hbm.at[dst_t.at[gn]], db, sd)
        return 0

    lax.fori_loop(0, NG // 2, _pair, 0)

    for sl in range(2):
        zb, sz, _ = zslots[sl]
        db, sd = dslots[sl]
        pltpu.make_async_copy(z_hbm.at[src_t.at[NG - 1]], zb, sz).wait()
        pltpu.make_async_copy(d_hbm.at[dst_t.at[NG - 1]], db, sd).wait()
    plsc.subcore_barrier()

    pltpu.sync_copy(
        acc_sh.at[pl.ds(s * rows_per_w, rows_per_w)],
        out_hbm.at[c, pl.ds(s * rows_per_w, rows_per_w)],
    )
    pltpu.sync_copy(den_t, den_hbm.at[wid])


def _fused_pass(z, d, src_r, dst_r):
    kern = pl.kernel(
        _fused_body,
        out_type=(
            jax.ShapeDtypeStruct((NW, N_NODES), jnp.float32),
            jax.ShapeDtypeStruct((NC, N_NODES, OUT_DIM), jnp.float32),
        ),
        mesh=_sc_mesh(),
        compiler_params=_SC_PARAMS,
        scratch_types=[
            pltpu.VMEM((NG, GRP), jnp.int32),
            pltpu.VMEM((NG, GRP), jnp.int32),
            pltpu.VMEM((N_NODES,), jnp.float32),
            pltpu.VMEM((GRP, OUT_DIM), jnp.float32),
            pltpu.VMEM((GRP, OUT_DIM), jnp.float32),
            pltpu.VMEM((GRP, OUT_DIM), jnp.float32),
            pltpu.VMEM((GRP, OUT_DIM), jnp.float32),
            pltpu.VMEM_SHARED((N_NODES, OUT_DIM), jnp.float32),
        ] + [pltpu.SemaphoreType.DMA] * 6,
    )
    return kern(z, d, src_r, dst_r)


def kernel(h, feat, edge_index, W_fc, W_dstfeat):
    src = edge_index[0]
    dst = edge_index[1]
    pad = EPAD - E_REAL
    src_r = jnp.pad(src, (0, pad)).reshape(NW, NG, GRP)
    dst_r = jnp.pad(dst, (0, pad)).reshape(NW, NG, GRP)

    z, d = _proj(h, feat, W_fc, W_dstfeat)
    den_parts, parts = _fused_pass(z, d, src_r, dst_r)
    return _combine(parts, den_parts)
